# baseline (speedup 1.0000x reference)
import functools

import jax
import jax.numpy as jnp
from jax import lax
from jax.experimental import pallas as pl
from jax.experimental.pallas import tpu as pltpu
from jax.experimental.pallas import tpu_sc as plsc

F = 8
N = 2048
P = 2048
B = 256

ROW_TILE = 256

NC = 2
NS = 16
L = 16
NW = NC * NS
RPW = N // NW
CH = 8
NCH = RPW // CH


def _sigmoid(x):
    return 1.0 / (1.0 + jnp.exp(-x))


def _combine_body(zst_ref, fcw_ref, fcb_ref, t_ref, out_ref):
    s = jnp.sum(zst_ref[...] * fcw_ref[...], axis=1, keepdims=True)
    s = s + fcb_ref[...]
    s = s - jnp.max(s)
    e = jnp.exp(s)
    w = e / jnp.sum(e)
    t = t_ref[...]
    acc = w[0, 0] * _sigmoid(t[0])
    for f in range(1, F):
        acc = acc + w[f, 0] * _sigmoid(t[f])
    out_ref[...] = acc.reshape(ROW_TILE * (P // 128), 128)


def _combine(zst, fc_w, fcb, factors):
    grid = (P // ROW_TILE,)
    return pl.pallas_call(
        _combine_body,
        grid=grid,
        in_specs=[
            pl.BlockSpec((F, B), lambda i: (0, 0)),
            pl.BlockSpec((1, B), lambda i: (0, 0)),
            pl.BlockSpec((1, 1), lambda i: (0, 0)),
            pl.BlockSpec((F, ROW_TILE, P), lambda i: (0, i, 0)),
        ],
        out_specs=pl.BlockSpec((ROW_TILE * (P // 128), 128), lambda i: (i, 0)),
        out_shape=jax.ShapeDtypeStruct((P * (P // 128), 128), jnp.float32),
    )(zst, fc_w, fcb, factors)


def _gather_body(m_hbm, vs_hbm, out_hbm, vs_v, idx_v, rowidx_v, rows_v, out_v,
                 sem_g0, sem_g1, sem_o0, sem_o1):
    wid = lax.axis_index("s") * NC + lax.axis_index("c")
    base = wid * RPW
    sem_g = (sem_g0, sem_g1)
    sem_o = (sem_o0, sem_o1)

    pltpu.sync_copy(vs_hbm, vs_v)

    def idx_body(c, carry):
        v = vs_v[pl.ds(c * L, L)]
        iv = (v * float(P)).astype(jnp.int32)
        iv = jnp.minimum(jnp.maximum(iv, 0), P - 1)
        idx_v[pl.ds(c * L, L)] = iv
        return carry

    lax.fori_loop(0, N // L, idx_body, 0)

    iota_c = lax.iota(jnp.int32, L)

    def ridx_body(j, carry):
        bc = plsc.load_gather(idx_v, [jnp.full((L,), base + j, jnp.int32)])
        rowidx_v[pl.ds(j * L, L)] = bc * 16 + iota_c
        return carry

    lax.fori_loop(0, RPW, ridx_body, 0)

    def start_gather(ch, b):
        return pltpu.async_copy(
            m_hbm.at[rowidx_v.at[pl.ds(ch * CH * 16, CH * 16)]],
            rows_v.at[b], sem_g[b])

    gat = [None, None]
    out = [None, None]
    gat[0] = start_gather(0, 0)

    for ch in range(NCH):
        b = ch % 2
        row0 = base + ch * CH
        gat[b].wait()
        if ch + 1 < NCH:
            gat[1 - b] = start_gather(ch + 1, 1 - b)
        if out[b] is not None:
            out[b].wait()
        rows_b = rows_v.at[b]
        out_rows = [out_v.at[b].at[r] for r in range(CH)]

        def col_body(jc, carry):
            joff = jc * L
            cidx = idx_v[pl.ds(joff, L)]
            hi = lax.shift_right_logical(cidx, 7)
            lo = lax.bitwise_and(cidx, 127)
            gs = [plsc.load_gather(rows_b, [hi + (r * 16), lo])
                  for r in range(CH)]
            for r in range(CH):
                out_rows[r][pl.ds(joff, L)] = gs[r]
            return carry

        lax.fori_loop(0, N // L, col_body, 0)
        out[b] = pltpu.async_copy(
            out_v.at[b], out_hbm.at[pl.ds(row0, CH)], sem_o[b])

    for b in range(2):
        if out[b] is not None:
            out[b].wait()


@functools.cache
def _gather():
    return pl.kernel(
        _gather_body,
        out_type=jax.ShapeDtypeStruct((N, N), jnp.float32),
        mesh=plsc.VectorSubcoreMesh(core_axis_name="c", subcore_axis_name="s"),
        scratch_types=[
            pltpu.VMEM((N,), jnp.float32),
            pltpu.VMEM((N,), jnp.int32),
            pltpu.VMEM((RPW * 16,), jnp.int32),
            pltpu.VMEM((2, CH * 16, 128), jnp.float32),
            pltpu.VMEM((2, CH, N), jnp.float32),
            pltpu.SemaphoreType.DMA,
            pltpu.SemaphoreType.DMA,
            pltpu.SemaphoreType.DMA,
            pltpu.SemaphoreType.DMA,
        ],
        compiler_params=pltpu.CompilerParams(
            use_tc_tiling_on_sc=False, needs_layout_passes=False
        ),
    )


def kernel(zs, vs, factors_graphon, fc_w, fc_b):
    zst = zs.T
    fcb = fc_b.reshape(1, 1)
    m = _combine(zst, fc_w, fcb, factors_graphon)
    return _gather()(m, vs)


# speedup vs baseline: 1.0016x; 1.0016x over previous
import functools

import jax
import jax.numpy as jnp
from jax import lax
from jax.experimental import pallas as pl
from jax.experimental.pallas import tpu as pltpu
from jax.experimental.pallas import tpu_sc as plsc

F = 8
N = 2048
P = 2048
B = 256

ROW_TILE = 128

NC = 2
NS = 16
L = 16
NW = NC * NS
RPW = N // NW
CH = 8
NCH = RPW // CH


def _sigmoid(x):
    return 1.0 / (1.0 + jnp.exp(-x))


def _combine_body(zst_ref, fcw_ref, fcb_ref, t_ref, out_ref):
    s = jnp.sum(zst_ref[...] * fcw_ref[...], axis=1, keepdims=True)
    s = s + fcb_ref[...]
    s = s - jnp.max(s)
    e = jnp.exp(s)
    w = e / jnp.sum(e)
    t = t_ref[...]
    acc = w[0, 0] * _sigmoid(t[0])
    for f in range(1, F):
        acc = acc + w[f, 0] * _sigmoid(t[f])
    out_ref[...] = acc.reshape(ROW_TILE * (P // 128), 128)


def _combine(zst, fc_w, fcb, factors):
    grid = (P // ROW_TILE,)
    return pl.pallas_call(
        _combine_body,
        grid=grid,
        in_specs=[
            pl.BlockSpec((F, B), lambda i: (0, 0)),
            pl.BlockSpec((1, B), lambda i: (0, 0)),
            pl.BlockSpec((1, 1), lambda i: (0, 0)),
            pl.BlockSpec((F, ROW_TILE, P), lambda i: (0, i, 0)),
        ],
        out_specs=pl.BlockSpec((ROW_TILE * (P // 128), 128), lambda i: (i, 0)),
        out_shape=jax.ShapeDtypeStruct((P * (P // 128), 128), jnp.float32),
    )(zst, fc_w, fcb, factors)


def _gather_body(m_hbm, vs_hbm, out_hbm, vs_v, idx_v, rowidx_v, rows_v, out_v,
                 sem_g0, sem_g1, sem_o0, sem_o1):
    wid = lax.axis_index("s") * NC + lax.axis_index("c")
    base = wid * RPW
    sem_g = (sem_g0, sem_g1)
    sem_o = (sem_o0, sem_o1)

    pltpu.sync_copy(vs_hbm, vs_v)

    def idx_body(c, carry):
        v = vs_v[pl.ds(c * L, L)]
        iv = (v * float(P)).astype(jnp.int32)
        iv = jnp.minimum(jnp.maximum(iv, 0), P - 1)
        idx_v[pl.ds(c * L, L)] = iv
        return carry

    lax.fori_loop(0, N // L, idx_body, 0)

    iota_c = lax.iota(jnp.int32, L)

    def ridx_body(j, carry):
        bc = plsc.load_gather(idx_v, [jnp.full((L,), base + j, jnp.int32)])
        rowidx_v[pl.ds(j * L, L)] = bc * 16 + iota_c
        return carry

    lax.fori_loop(0, RPW, ridx_body, 0)

    def start_gather(ch, b):
        return pltpu.async_copy(
            m_hbm.at[rowidx_v.at[pl.ds(ch * CH * 16, CH * 16)]],
            rows_v.at[b], sem_g[b])

    gat = [None, None]
    out = [None, None]
    gat[0] = start_gather(0, 0)

    for ch in range(NCH):
        b = ch % 2
        row0 = base + ch * CH
        gat[b].wait()
        if ch + 1 < NCH:
            gat[1 - b] = start_gather(ch + 1, 1 - b)
        if out[b] is not None:
            out[b].wait()
        rows_b = rows_v.at[b]
        out_rows = [out_v.at[b].at[r] for r in range(CH)]

        def col_body(jc, carry):
            joff = jc * L
            cidx = idx_v[pl.ds(joff, L)]
            hi = lax.shift_right_logical(cidx, 7)
            lo = lax.bitwise_and(cidx, 127)
            gs = [plsc.load_gather(rows_b, [hi + (r * 16), lo])
                  for r in range(CH)]
            for r in range(CH):
                out_rows[r][pl.ds(joff, L)] = gs[r]
            return carry

        lax.fori_loop(0, N // L, col_body, 0)
        out[b] = pltpu.async_copy(
            out_v.at[b], out_hbm.at[pl.ds(row0, CH)], sem_o[b])

    for b in range(2):
        if out[b] is not None:
            out[b].wait()


@functools.cache
def _gather():
    return pl.kernel(
        _gather_body,
        out_type=jax.ShapeDtypeStruct((N, N), jnp.float32),
        mesh=plsc.VectorSubcoreMesh(core_axis_name="c", subcore_axis_name="s"),
        scratch_types=[
            pltpu.VMEM((N,), jnp.float32),
            pltpu.VMEM((N,), jnp.int32),
            pltpu.VMEM((RPW * 16,), jnp.int32),
            pltpu.VMEM((2, CH * 16, 128), jnp.float32),
            pltpu.VMEM((2, CH, N), jnp.float32),
            pltpu.SemaphoreType.DMA,
            pltpu.SemaphoreType.DMA,
            pltpu.SemaphoreType.DMA,
            pltpu.SemaphoreType.DMA,
        ],
        compiler_params=pltpu.CompilerParams(
            use_tc_tiling_on_sc=False, needs_layout_passes=False
        ),
    )


def kernel(zs, vs, factors_graphon, fc_w, fc_b):
    zst = zs.T
    fcb = fc_b.reshape(1, 1)
    m = _combine(zst, fc_w, fcb, factors_graphon)
    return _gather()(m, vs)
